# Initial kernel scaffold; baseline (speedup 1.0000x reference)
import jax
import jax.numpy as jnp
from jax.experimental import pallas as pl


def kernel(x, edge_index, c0_W, c0_b, c1_W, c1_b, c2_W, c2_b, ln_g, ln_b, proj_W, proj_b):
    raise NotImplementedError("write your pallas kernel here")


# speedup vs baseline: 4.6953x; 4.6953x over previous
import functools

import jax
import jax.numpy as jnp
from jax import lax
from jax.experimental import pallas as pl
from jax.experimental.pallas import tpu as pltpu
from jax.experimental.pallas import tpu_sc as plsc

_NC = 2
_NS = 16
_NW = _NC * _NS
_GRP = 128
_BR = 1000


def _mesh():
    return plsc.VectorSubcoreMesh(core_axis_name="c", subcore_axis_name="s",
                                  num_cores=_NC, num_subcores=_NS)


@functools.lru_cache(maxsize=None)
def _build_prop(n_acc, gpw, h):
    rpt = n_acc // _NS

    @functools.partial(
        pl.kernel,
        out_type=jax.ShapeDtypeStruct((_NC, n_acc, h), jnp.float32),
        mesh=_mesh(),
        scratch_types=[
            pltpu.VMEM((_GRP,), jnp.int32),
            pltpu.VMEM((_GRP,), jnp.int32),
            pltpu.VMEM((_GRP, h), jnp.float32),
            pltpu.VMEM((16, h), jnp.float32),
            pltpu.VMEM_SHARED((n_acc, h), jnp.float32),
            pltpu.SemaphoreType.DMA,
        ],
    )
    def prop(xs_hbm, srcp_hbm, dstp_hbm, out_hbm, idx_s, idx_d, rows, zbuf,
             acc, sem):
        c = lax.axis_index("c")
        s = lax.axis_index("s")
        wid = c * _NS + s
        zero16 = jnp.zeros((16,), jnp.float32)
        for i in range(16):
            for j in range(h // 16):
                zbuf[i, pl.ds(j * 16, 16)] = zero16
        base_row = s * rpt
        for j in range(rpt // 16):
            pltpu.sync_copy(zbuf, acc.at[pl.ds(base_row + j * 16, 16)])
        plsc.subcore_barrier()

        def body(g, carry):
            pltpu.sync_copy(srcp_hbm.at[wid, g], idx_s)
            pltpu.sync_copy(dstp_hbm.at[wid, g], idx_d)
            pltpu.async_copy(xs_hbm.at[idx_s], rows, sem).wait()
            pltpu.sync_copy(rows, acc.at[idx_d], add=True)
            return carry

        lax.fori_loop(0, gpw, body, 0)
        plsc.subcore_barrier()
        pltpu.sync_copy(acc.at[pl.ds(base_row, rpt)],
                        out_hbm.at[c, pl.ds(base_row, rpt)])

    return prop


@functools.lru_cache(maxsize=None)
def _build_hist(n_acc, gpw):
    rpt = n_acc // _NS

    @functools.partial(
        pl.kernel,
        out_type=jax.ShapeDtypeStruct((_NC, n_acc), jnp.float32),
        mesh=_mesh(),
        scratch_types=[
            pltpu.VMEM((_GRP,), jnp.int32),
            pltpu.VMEM((_GRP,), jnp.float32),
            pltpu.VMEM((rpt,), jnp.float32),
            pltpu.VMEM_SHARED((n_acc,), jnp.float32),
        ],
    )
    def hist(dstp_hbm, out_hbm, idx_d, ones, zrow, acc):
        c = lax.axis_index("c")
        s = lax.axis_index("s")
        wid = c * _NS + s
        one16 = jnp.ones((16,), jnp.float32)
        zero16 = jnp.zeros((16,), jnp.float32)
        for j in range(_GRP // 16):
            ones[pl.ds(j * 16, 16)] = one16
        for j in range(rpt // 16):
            zrow[pl.ds(j * 16, 16)] = zero16
        pltpu.sync_copy(zrow, acc.at[pl.ds(s * rpt, rpt)])
        plsc.subcore_barrier()

        def body(g, carry):
            pltpu.sync_copy(dstp_hbm.at[wid, g], idx_d)
            pltpu.sync_copy(ones, acc.at[idx_d], add=True)
            return carry

        lax.fori_loop(0, gpw, body, 0)
        plsc.subcore_barrier()
        pltpu.sync_copy(acc.at[pl.ds(s * rpt, rpt)],
                        out_hbm.at[c, pl.ds(s * rpt, rpt)])

    return hist


def _dinv_kernel(degp):
    def body(d_ref, o_ref):
        deg = d_ref[0] + d_ref[1]
        o_ref[...] = jnp.where(deg > 0.0, lax.rsqrt(deg), 0.0)

    return pl.pallas_call(
        body,
        out_shape=jax.ShapeDtypeStruct(degp.shape[1:], jnp.float32),
    )(degp)


def _tag_start(h, dn, w):
    n, hh = h.shape
    ho = w.shape[1]

    def body(h_ref, d_ref, w_ref, out_ref, t_ref):
        hv = h_ref[...]
        out_ref[...] = jnp.dot(hv, w_ref[...],
                               preferred_element_type=jnp.float32)
        t_ref[...] = hv * d_ref[...]

    return pl.pallas_call(
        body,
        grid=(n // _BR,),
        in_specs=[
            pl.BlockSpec((_BR, hh), lambda i: (i, 0)),
            pl.BlockSpec((_BR, 1), lambda i: (i, 0)),
            pl.BlockSpec((hh, ho), lambda i: (0, 0)),
        ],
        out_specs=[pl.BlockSpec((_BR, ho), lambda i: (i, 0)),
                   pl.BlockSpec((_BR, hh), lambda i: (i, 0))],
        out_shape=[jax.ShapeDtypeStruct((n, ho), jnp.float32),
                   jax.ShapeDtypeStruct((n, hh), jnp.float32)],
    )(h, dn, w)


def _xk(p_ref, d_ref):
    return (p_ref[0] + p_ref[1]) * d_ref[...]


def _tag_step(out_in, p, dn, w):
    n, hh = out_in.shape

    def body(o_ref, p_ref, d_ref, w_ref, out_ref, t_ref):
        xk = _xk(p_ref, d_ref)
        out_ref[...] = o_ref[...] + jnp.dot(xk, w_ref[...],
                                            preferred_element_type=jnp.float32)
        t_ref[...] = xk * d_ref[...]

    return pl.pallas_call(
        body,
        grid=(n // _BR,),
        in_specs=[
            pl.BlockSpec((_BR, hh), lambda i: (i, 0)),
            pl.BlockSpec((2, _BR, hh), lambda i: (0, i, 0)),
            pl.BlockSpec((_BR, 1), lambda i: (i, 0)),
            pl.BlockSpec((hh, hh), lambda i: (0, 0)),
        ],
        out_specs=[pl.BlockSpec((_BR, hh), lambda i: (i, 0))] * 2,
        out_shape=[jax.ShapeDtypeStruct((n, hh), jnp.float32)] * 2,
    )(out_in, p, dn, w)


def _tag_fin0(out_in, p, dn, w, b, g, lb):
    n, hh = out_in.shape

    def body(o_ref, p_ref, d_ref, w_ref, b_ref, g_ref, lb_ref, out_ref):
        xk = _xk(p_ref, d_ref)
        o = o_ref[...] + jnp.dot(xk, w_ref[...],
                                 preferred_element_type=jnp.float32)
        o = o + b_ref[...]
        mu = jnp.mean(o, axis=-1, keepdims=True)
        var = jnp.mean((o - mu) ** 2, axis=-1, keepdims=True)
        o = (o - mu) * lax.rsqrt(var + 1e-5) * g_ref[...] + lb_ref[...]
        out_ref[...] = jnp.maximum(o, 0.0)

    return pl.pallas_call(
        body,
        grid=(n // _BR,),
        in_specs=[
            pl.BlockSpec((_BR, hh), lambda i: (i, 0)),
            pl.BlockSpec((2, _BR, hh), lambda i: (0, i, 0)),
            pl.BlockSpec((_BR, 1), lambda i: (i, 0)),
            pl.BlockSpec((hh, hh), lambda i: (0, 0)),
            pl.BlockSpec((1, hh), lambda i: (0, 0)),
            pl.BlockSpec((1, hh), lambda i: (0, 0)),
            pl.BlockSpec((1, hh), lambda i: (0, 0)),
        ],
        out_specs=pl.BlockSpec((_BR, hh), lambda i: (i, 0)),
        out_shape=jax.ShapeDtypeStruct((n, hh), jnp.float32),
    )(out_in, p, dn, w, b, g, lb)


def _tag_fin1(out_in, p, dn, w, b, hskip):
    n, hh = out_in.shape

    def body(o_ref, p_ref, d_ref, w_ref, b_ref, s_ref, out_ref):
        xk = _xk(p_ref, d_ref)
        o = o_ref[...] + jnp.dot(xk, w_ref[...],
                                 preferred_element_type=jnp.float32)
        o = o + b_ref[...]
        out_ref[...] = jnp.maximum(o, 0.0) + s_ref[...]

    return pl.pallas_call(
        body,
        grid=(n // _BR,),
        in_specs=[
            pl.BlockSpec((_BR, hh), lambda i: (i, 0)),
            pl.BlockSpec((2, _BR, hh), lambda i: (0, i, 0)),
            pl.BlockSpec((_BR, 1), lambda i: (i, 0)),
            pl.BlockSpec((hh, hh), lambda i: (0, 0)),
            pl.BlockSpec((1, hh), lambda i: (0, 0)),
            pl.BlockSpec((_BR, hh), lambda i: (i, 0)),
        ],
        out_specs=pl.BlockSpec((_BR, hh), lambda i: (i, 0)),
        out_shape=jax.ShapeDtypeStruct((n, hh), jnp.float32),
    )(out_in, p, dn, w, b, hskip)


def _tag_fin2(out_in, p, dn, w, b, pw, pb):
    n, hh = out_in.shape
    dm = pw.shape[1]

    def body(o_ref, p_ref, d_ref, w_ref, b_ref, pw_ref, pb_ref, out_ref):
        xk = _xk(p_ref, d_ref)
        o = o_ref[...] + jnp.dot(xk, w_ref[...],
                                 preferred_element_type=jnp.float32)
        o = o + b_ref[...]
        out_ref[...] = jnp.dot(o, pw_ref[...],
                               preferred_element_type=jnp.float32) + pb_ref[...]

    return pl.pallas_call(
        body,
        grid=(n // _BR,),
        in_specs=[
            pl.BlockSpec((_BR, hh), lambda i: (i, 0)),
            pl.BlockSpec((2, _BR, hh), lambda i: (0, i, 0)),
            pl.BlockSpec((_BR, 1), lambda i: (i, 0)),
            pl.BlockSpec((hh, hh), lambda i: (0, 0)),
            pl.BlockSpec((1, hh), lambda i: (0, 0)),
            pl.BlockSpec((hh, dm), lambda i: (0, 0)),
            pl.BlockSpec((1, dm), lambda i: (0, 0)),
        ],
        out_specs=pl.BlockSpec((_BR, dm), lambda i: (i, 0)),
        out_shape=jax.ShapeDtypeStruct((n, dm), jnp.float32),
    )(out_in, p, dn, w, b, pw, pb)


def kernel(x, edge_index, c0_W, c0_b, c1_W, c1_b, c2_W, c2_b, ln_g, ln_b,
           proj_W, proj_b):
    n, _ = x.shape
    kk = c0_W.shape[0] - 1
    e = edge_index.shape[1]

    gpw = -(-e // (_NW * _GRP))
    ep = _NW * gpw * _GRP
    n_acc = 256 * (-(-(n + 1) // 256))
    dummy = n

    src = edge_index[0]
    dst = edge_index[1]
    srcp = jnp.concatenate(
        [src, jnp.zeros((ep - e,), jnp.int32)]).reshape(_NW, gpw, _GRP)
    dstp = jnp.concatenate(
        [dst, jnp.full((ep - e,), dummy, jnp.int32)]).reshape(_NW, gpw, _GRP)

    hist = _build_hist(n_acc, gpw)
    prop = _build_prop(n_acc, gpw, x.shape[1])

    degp = hist(dstp)
    dinv = _dinv_kernel(degp.reshape(2, n_acc, 1))
    dn = dinv[:n]

    def run_tag(hcur, w_all, fin):
        out, t = _tag_start(hcur, dn, w_all[0])
        for k in range(1, kk):
            p = prop(t, srcp, dstp)
            out, t = _tag_step(out, p, dn, w_all[k])
        p = prop(t, srcp, dstp)
        return fin(out, p)

    h0 = run_tag(x, c0_W,
                 lambda o, p: _tag_fin0(o, p, dn, c0_W[kk], c0_b[None, :],
                                        ln_g[None, :], ln_b[None, :]))
    h1 = run_tag(h0, c1_W,
                 lambda o, p: _tag_fin1(o, p, dn, c1_W[kk], c1_b[None, :], h0))
    return run_tag(h1, c2_W,
                   lambda o, p: _tag_fin2(o, p, dn, c2_W[kk], c2_b[None, :],
                                          proj_W, proj_b[None, :]))


# speedup vs baseline: 12.5128x; 2.6650x over previous
import functools

import jax
import jax.numpy as jnp
from jax import lax
from jax.experimental import pallas as pl
from jax.experimental.pallas import tpu as pltpu
from jax.experimental.pallas import tpu_sc as plsc

_NC = 2
_NS = 16
_NW = _NC * _NS
_GRP = 128
_BR = 1000


def _mesh():
    return plsc.VectorSubcoreMesh(core_axis_name="c", subcore_axis_name="s",
                                  num_cores=_NC, num_subcores=_NS)


@functools.lru_cache(maxsize=None)
def _build_layer_prop(n_acc, gpt, h, kk):
    hh = h // 2
    rpt = n_acc // _NS
    cs = 16
    ncc = gpt // cs

    @functools.partial(
        pl.kernel,
        out_type=jax.ShapeDtypeStruct((kk, _NC, n_acc, hh), jnp.float32),
        mesh=_mesh(),
        compiler_params=pltpu.CompilerParams(use_tc_tiling_on_sc=False),
        scratch_types=[
            pltpu.VMEM((cs, _GRP), jnp.int32),
            pltpu.VMEM((cs, _GRP), jnp.int32),
            pltpu.VMEM((cs, _GRP), jnp.int32),
            pltpu.VMEM((cs, _GRP), jnp.int32),
            pltpu.VMEM((_GRP, hh), jnp.float32),
            pltpu.VMEM((_GRP, hh), jnp.float32),
            pltpu.VMEM((_GRP, hh), jnp.float32),
            pltpu.VMEM((_GRP, hh), jnp.float32),
            pltpu.VMEM((rpt + 16,), jnp.float32),
            pltpu.VMEM_SHARED((n_acc, hh), jnp.float32),
            pltpu.VMEM_SHARED((n_acc, hh), jnp.float32),
            pltpu.SemaphoreType.DMA,
            pltpu.SemaphoreType.DMA,
            pltpu.SemaphoreType.DMA,
            pltpu.SemaphoreType.DMA,
            pltpu.SemaphoreType.DMA,
            pltpu.SemaphoreType.DMA,
            pltpu.SemaphoreType.DMA,
            pltpu.SemaphoreType.DMA,
            pltpu.SemaphoreType.DMA,
            pltpu.SemaphoreType.DMA,
            pltpu.SemaphoreType.DMA,
        ],
    )
    def prop(xs_hbm, srcp_hbm, dstp_hbm, dinv_hbm, out_hbm,
             idx_s0, idx_s1, idx_d0, idx_d1, rows0, rows1, rows2, rows3,
             dinvb, bufa, bufb, sem0, sem1, sem2, sem3, isem0, isem1, ssem,
             ssm0, ssm1, ssm2, ssm3):
        c = lax.axis_index("c")
        s = lax.axis_index("s")
        base_row = s * rpt
        cstage = pltpu.async_copy(
            xs_hbm.at[pl.ds(c * n_acc + base_row, rpt)],
            bufa.at[pl.ds(base_row, rpt)], ssem)
        cdinv = pltpu.async_copy(dinv_hbm.at[pl.ds(base_row, rpt + 16)],
                                 dinvb, ssem)
        pltpu.async_copy(srcp_hbm.at[s, pl.ds(0, cs)], idx_s0, isem0)
        pltpu.async_copy(dstp_hbm.at[s, pl.ds(0, cs)], idx_d0, isem0)
        pltpu.async_copy(srcp_hbm.at[s, pl.ds(cs, cs)], idx_s1, isem1)
        pltpu.async_copy(dstp_hbm.at[s, pl.ds(cs, cs)], idx_d1, isem1)
        zero16 = jnp.zeros((16,), jnp.float32)

        def zero_rows0():
            for i in range(_GRP):
                for j in range(hh // 16):
                    rows0[i, pl.ds(j * 16, 16)] = zero16

        def zero_stripe(buf):
            for j in range(rpt // _GRP):
                pltpu.sync_copy(rows0,
                                buf.at[pl.ds(base_row + j * _GRP, _GRP)])

        zero_rows0()
        zero_stripe(bufb)
        cstage.wait()
        cdinv.wait()
        plsc.subcore_barrier()

        rows = (rows0, rows1, rows2, rows3)
        sems = (sem0, sem1, sem2, sem3)
        ssms = (ssm0, ssm1, ssm2, ssm3)
        idx_sb = (idx_s0, idx_s1)
        idx_db = (idx_d0, idx_d1)
        isems = (isem0, isem1)

        for j in range(kk):
            table = bufa if j % 2 == 0 else bufb
            acc = bufb if j % 2 == 0 else bufa

            def chunk_pair(ccp, carry):
                for par in range(2):
                    cc = 2 * ccp + par
                    ixs, ixd, ism = idx_sb[par], idx_db[par], isems[par]
                    pltpu.make_async_copy(srcp_hbm.at[s, pl.ds(0, cs)], ixs,
                                          ism).wait()
                    pltpu.make_async_copy(dstp_hbm.at[s, pl.ds(0, cs)], ixd,
                                          ism).wait()
                    gcps = [None] * 4
                    scps = [None] * 4
                    for g in range(2):
                        gcps[g] = pltpu.async_copy(table.at[ixs.at[g]],
                                                   rows[g], sems[g])
                    for g in range(cs):
                        b = g % 4
                        bn = (g + 2) % 4
                        gcps[b].wait()
                        scps[b] = pltpu.async_copy(
                            rows[b], acc.at[ixd.at[g]], ssms[b], add=True)
                        if g + 2 < cs:
                            if g >= 2:
                                scps[bn].wait()
                            gcps[bn] = pltpu.async_copy(
                                table.at[ixs.at[g + 2]], rows[bn], sems[bn])
                    for g in range(cs - 4, cs):
                        scps[g % 4].wait()
                    nxt = lax.rem(cc + 2, ncc) * cs
                    pltpu.async_copy(srcp_hbm.at[s, pl.ds(nxt, cs)], ixs, ism)
                    pltpu.async_copy(dstp_hbm.at[s, pl.ds(nxt, cs)], ixd, ism)
                return carry

            lax.fori_loop(0, ncc // 2, chunk_pair, 0)
            plsc.subcore_barrier()
            if j < kk - 1:
                def trans(u, carry):
                    rs = base_row + u * _GRP
                    pltpu.sync_copy(acc.at[pl.ds(rs, _GRP)], rows1)
                    pltpu.async_copy(
                        rows1, out_hbm.at[j, c, pl.ds(rs, _GRP)],
                        ssem).wait()

                    def srow(r, carry2):
                        dv = dinvb[pl.ds(u * _GRP + r, 16)][0]
                        dv2 = dv * dv
                        for jj in range(hh // 16):
                            v = rows1[r, pl.ds(jj * 16, 16)]
                            rows1[r, pl.ds(jj * 16, 16)] = v * dv2
                        return carry2

                    lax.fori_loop(0, _GRP, srow, 0)
                    pltpu.sync_copy(rows1, acc.at[pl.ds(rs, _GRP)])
                    return carry

                lax.fori_loop(0, rpt // _GRP, trans, 0)
                zero_rows0()
                zero_stripe(table)
                plsc.subcore_barrier()
            else:
                pltpu.sync_copy(acc.at[pl.ds(base_row, rpt)],
                                out_hbm.at[j, c, pl.ds(base_row, rpt)])
        for par in range(2):
            pltpu.make_async_copy(srcp_hbm.at[s, pl.ds(0, cs)],
                                  idx_sb[par], isems[par]).wait()
            pltpu.make_async_copy(dstp_hbm.at[s, pl.ds(0, cs)],
                                  idx_db[par], isems[par]).wait()

    return prop


@functools.lru_cache(maxsize=None)
def _build_hist(n_acc, gpt):
    gph = gpt // 2
    rpt = n_acc // _NS

    @functools.partial(
        pl.kernel,
        out_type=jax.ShapeDtypeStruct((_NC, n_acc), jnp.float32),
        mesh=_mesh(),
        scratch_types=[
            pltpu.VMEM((gph, _GRP), jnp.int32),
            pltpu.VMEM((_GRP,), jnp.float32),
            pltpu.VMEM((rpt,), jnp.float32),
            pltpu.VMEM_SHARED((n_acc,), jnp.float32),
            pltpu.SemaphoreType.DMA,
            pltpu.SemaphoreType.DMA,
            pltpu.SemaphoreType.DMA,
        ],
    )
    def hist(dstp_hbm, out_hbm, idx_d, ones, zrow, acc, sem0, sem1, isem):
        c = lax.axis_index("c")
        s = lax.axis_index("s")
        cid = pltpu.async_copy(dstp_hbm.at[s, pl.ds(c * gph, gph)], idx_d,
                               isem)
        zero16 = jnp.zeros((16,), jnp.float32)
        one16 = jnp.ones((16,), jnp.float32)
        for j in range(_GRP // 16):
            ones[pl.ds(j * 16, 16)] = one16
        for j in range(rpt // 16):
            zrow[pl.ds(j * 16, 16)] = zero16
        pltpu.sync_copy(zrow, acc.at[pl.ds(s * rpt, rpt)])
        cid.wait()
        plsc.subcore_barrier()

        def pair(gg, carry):
            g0 = 2 * gg
            cp0 = pltpu.async_copy(ones, acc.at[idx_d.at[g0]], sem0, add=True)
            cp1 = pltpu.async_copy(ones, acc.at[idx_d.at[g0 + 1]], sem1,
                                   add=True)
            cp0.wait()
            cp1.wait()
            return carry

        lax.fori_loop(0, gph // 2, pair, 0)
        plsc.subcore_barrier()
        pltpu.sync_copy(acc.at[pl.ds(s * rpt, rpt)],
                        out_hbm.at[c, pl.ds(s * rpt, rpt)])

    return hist


def _dinv_kernel(degp):
    def body(d_ref, o_ref):
        deg = d_ref[0] + d_ref[1]
        o_ref[...] = jnp.where(deg > 0.0, lax.rsqrt(deg), 0.0)

    return pl.pallas_call(
        body,
        out_shape=jax.ShapeDtypeStruct(degp.shape[1:], jnp.float32),
    )(degp)


def _store_halves(t_ref, th, hh):
    t_ref[0] = th[:, : hh // 2]
    t_ref[1] = th[:, hh // 2:]


def _tag_start(h, dn, w):
    n, hh = h.shape
    n_acc = 256 * (-(-(n + 1) // 256))
    ho = w.shape[1]

    def body(h_ref, d_ref, w_ref, out_ref, t_ref):
        hv = h_ref[...]
        out_ref[...] = jnp.dot(hv, w_ref[...],
                               preferred_element_type=jnp.float32)
        _store_halves(t_ref, hv * d_ref[...], hh)

    return pl.pallas_call(
        body,
        grid=(n // _BR,),
        in_specs=[
            pl.BlockSpec((_BR, hh), lambda i: (i, 0)),
            pl.BlockSpec((_BR, 1), lambda i: (i, 0)),
            pl.BlockSpec((hh, ho), lambda i: (0, 0)),
        ],
        out_specs=[pl.BlockSpec((_BR, ho), lambda i: (i, 0)),
                   pl.BlockSpec((2, _BR, hh // 2), lambda i: (0, i, 0))],
        out_shape=[jax.ShapeDtypeStruct((n, ho), jnp.float32),
                   jax.ShapeDtypeStruct((2, n_acc, hh // 2), jnp.float32)],
    )(h, dn, w)


def _xk(p_ref, k, d_ref):
    return jnp.concatenate([p_ref[k, 0], p_ref[k, 1]], axis=1) * d_ref[...]


def _acc_out(o_ref, p_ref, d_ref, w_ref, nk):
    o = o_ref[...]
    for k in range(nk):
        o = o + jnp.dot(_xk(p_ref, k, d_ref), w_ref[k],
                        preferred_element_type=jnp.float32)
    return o


def _tag_fin_start(out_in, p, dn, w, b, g, lb, hskip, w0n):
    n, hh = out_in.shape
    nk = w.shape[0]
    n_acc = p.shape[2]
    use_ln = hskip is None

    def body(o_ref, p_ref, d_ref, w_ref, b_ref, a_ref, b2_ref, w0_ref,
             h_ref, out_ref, t_ref):
        o = _acc_out(o_ref, p_ref, d_ref, w_ref, nk) + b_ref[...]
        if use_ln:
            mu = jnp.mean(o, axis=-1, keepdims=True)
            var = jnp.mean((o - mu) ** 2, axis=-1, keepdims=True)
            o = (o - mu) * lax.rsqrt(var + 1e-5) * a_ref[...] + b2_ref[...]
            h = jnp.maximum(o, 0.0)
        else:
            h = jnp.maximum(o, 0.0) + a_ref[...]
        h_ref[...] = h
        out_ref[...] = jnp.dot(h, w0_ref[...],
                               preferred_element_type=jnp.float32)
        _store_halves(t_ref, h * d_ref[...], hh)

    if use_ln:
        aux_a, aux_b = g, lb
        aspec = pl.BlockSpec((1, hh), lambda i: (0, 0))
    else:
        aux_a, aux_b = hskip, b
        aspec = pl.BlockSpec((_BR, hh), lambda i: (i, 0))

    return pl.pallas_call(
        body,
        grid=(n // _BR,),
        in_specs=[
            pl.BlockSpec((_BR, hh), lambda i: (i, 0)),
            pl.BlockSpec((nk, 2, _BR, hh // 2), lambda i: (0, 0, i, 0)),
            pl.BlockSpec((_BR, 1), lambda i: (i, 0)),
            pl.BlockSpec((nk, hh, hh), lambda i: (0, 0, 0)),
            pl.BlockSpec((1, hh), lambda i: (0, 0)),
            aspec,
            pl.BlockSpec((1, hh), lambda i: (0, 0)),
            pl.BlockSpec((hh, hh), lambda i: (0, 0)),
        ],
        out_specs=[pl.BlockSpec((_BR, hh), lambda i: (i, 0)),
                   pl.BlockSpec((_BR, hh), lambda i: (i, 0)),
                   pl.BlockSpec((2, _BR, hh // 2), lambda i: (0, i, 0))],
        out_shape=[jax.ShapeDtypeStruct((n, hh), jnp.float32),
                   jax.ShapeDtypeStruct((n, hh), jnp.float32),
                   jax.ShapeDtypeStruct((2, n_acc, hh // 2), jnp.float32)],
    )(out_in, p, dn, w, b, aux_a, aux_b, w0n)


def _tag_fin2(out_in, p, dn, w, b, pw, pb):
    n, hh = out_in.shape
    nk = w.shape[0]
    dm = pw.shape[1]

    def body(o_ref, p_ref, d_ref, w_ref, b_ref, pw_ref, pb_ref, out_ref):
        o = _acc_out(o_ref, p_ref, d_ref, w_ref, nk) + b_ref[...]
        out_ref[...] = jnp.dot(o, pw_ref[...],
                               preferred_element_type=jnp.float32) + pb_ref[...]

    return pl.pallas_call(
        body,
        grid=(n // _BR,),
        in_specs=[
            pl.BlockSpec((_BR, hh), lambda i: (i, 0)),
            pl.BlockSpec((nk, 2, _BR, hh // 2), lambda i: (0, 0, i, 0)),
            pl.BlockSpec((_BR, 1), lambda i: (i, 0)),
            pl.BlockSpec((nk, hh, hh), lambda i: (0, 0, 0)),
            pl.BlockSpec((1, hh), lambda i: (0, 0)),
            pl.BlockSpec((hh, dm), lambda i: (0, 0)),
            pl.BlockSpec((1, dm), lambda i: (0, 0)),
        ],
        out_specs=pl.BlockSpec((_BR, dm), lambda i: (i, 0)),
        out_shape=jax.ShapeDtypeStruct((n, dm), jnp.float32),
    )(out_in, p, dn, w, b, pw, pb)


def kernel(x, edge_index, c0_W, c0_b, c1_W, c1_b, c2_W, c2_b, ln_g, ln_b,
           proj_W, proj_b):
    n, _ = x.shape
    kk = c0_W.shape[0] - 1
    e = edge_index.shape[1]

    gpt = -(-e // (_NS * _GRP))
    gpt = gpt + (-gpt) % 32
    ep = _NS * gpt * _GRP
    n_acc = 256 * (-(-(n + 1) // 256))
    dummy = n

    src = edge_index[0]
    dst = edge_index[1]
    srcp = jnp.concatenate(
        [src, jnp.zeros((ep - e,), jnp.int32)]).reshape(_NS, gpt, _GRP)
    dstp = jnp.concatenate(
        [dst, jnp.full((ep - e,), dummy, jnp.int32)]).reshape(_NS, gpt, _GRP)

    hist = _build_hist(n_acc, gpt)
    prop = _build_layer_prop(n_acc, gpt, x.shape[1], kk)

    degp = hist(dstp)
    dinv = _dinv_kernel(degp.reshape(2, n_acc, 1))
    dn = dinv[:n]
    dinv_pad = jnp.concatenate(
        [dinv.reshape(n_acc), jnp.zeros((16,), jnp.float32)])

    out0, t = _tag_start(x, dn, c0_W[0])
    p = prop(t.reshape(2 * n_acc, -1), srcp, dstp, dinv_pad)
    h0, out1, t = _tag_fin_start(out0, p, dn, c0_W[1:], c0_b[None, :],
                                 ln_g[None, :], ln_b[None, :], None, c1_W[0])
    p = prop(t.reshape(2 * n_acc, -1), srcp, dstp, dinv_pad)
    _, out2, t = _tag_fin_start(out1, p, dn, c1_W[1:], c1_b[None, :],
                                None, None, h0, c2_W[0])
    p = prop(t.reshape(2 * n_acc, -1), srcp, dstp, dinv_pad)
    return _tag_fin2(out2, p, dn, c2_W[1:], c2_b[None, :],
                     proj_W, proj_b[None, :])
